# Initial kernel scaffold; baseline (speedup 1.0000x reference)
import jax
import jax.numpy as jnp
from jax.experimental import pallas as pl


def kernel(embed, edge_index, W1, b1, W2, b2, top_k):
    raise NotImplementedError("write your pallas kernel here")


# speedup vs baseline: 1.3646x; 1.3646x over previous
import functools

import jax
import jax.numpy as jnp
from jax import lax
from jax.experimental import pallas as pl
from jax.experimental.pallas import tpu as pltpu
from jax.experimental.pallas import tpu_sc as plsc

NC = 2
NS = 16
NW = NC * NS
LANES = 16

CH = 128


def _tables_body(x_ref, w1_ref, b1_ref, ga_ref, gb_ref):
    x = x_ref[...]
    d = x.shape[1]
    ga_ref[...] = jnp.dot(x, w1_ref[0:d, :], preferred_element_type=jnp.float32) + b1_ref[...]
    gb_ref[...] = jnp.dot(x, w1_ref[d : 2 * d, :], preferred_element_type=jnp.float32)


def _node_tables(embed, W1, b1):
    n, d = embed.shape
    hid = W1.shape[1]
    blk = 1000
    grid = n // blk
    return pl.pallas_call(
        _tables_body,
        grid=(grid,),
        in_specs=[
            pl.BlockSpec((blk, d), lambda i: (i, 0)),
            pl.BlockSpec((2 * d, hid), lambda i: (0, 0)),
            pl.BlockSpec((1, hid), lambda i: (0, 0)),
        ],
        out_specs=[
            pl.BlockSpec((blk, hid), lambda i: (i, 0)),
            pl.BlockSpec((blk, hid), lambda i: (i, 0)),
        ],
        out_shape=[
            jax.ShapeDtypeStruct((n, hid), jnp.float32),
            jax.ShapeDtypeStruct((n, hid), jnp.float32),
        ],
    )(embed, W1, b1.reshape(1, hid))


def _score_body(nchunks, nloop, col_hbm, row_hbm, ga_hbm, gb_hbm, w2p_hbm, em_hbm,
                colv, rowv, accA, accB, w2v, trb, emv, semA, semB):
    cid = lax.axis_index("c")
    sid = lax.axis_index("s")
    wid = sid * NC + cid

    pltpu.sync_copy(w2p_hbm, w2v)
    w2k = [w2v[pl.ds(16 * k, 16)] for k in range(4)]
    b2v = w2v[pl.ds(64, 16)]
    iota16x = lax.iota(jnp.int32, LANES) * LANES

    def chunk_body(i, _):
        c = i * NW + wid

        @pl.when(c < nchunks)
        def _():
            base = c * CH
            pltpu.sync_copy(col_hbm.at[pl.ds(base, CH)], colv)
            pltpu.sync_copy(row_hbm.at[pl.ds(base, CH)], rowv)
            cpA = pltpu.async_copy(ga_hbm.at[colv], accA, semA)
            cpB = pltpu.async_copy(gb_hbm.at[rowv], accB, semB)
            cpA.wait()
            cpB.wait()

            def group_body(g, _):
                e0 = g * LANES
                for j in range(LANES):
                    s = jnp.zeros((LANES,), jnp.float32)
                    for k in range(4):
                        z = accA[e0 + j, pl.ds(16 * k, 16)] + accB[e0 + j, pl.ds(16 * k, 16)]
                        s = s + jnp.maximum(z, 0.0) * w2k[k]
                    trb[pl.ds(j * LANES, LANES)] = s
                tot = b2v
                for cc in range(LANES):
                    tot = tot + plsc.load_gather(trb, [iota16x + cc])
                emv[pl.ds(e0, LANES)] = 1.0 / (1.0 + jnp.exp(-tot))
                return 0

            lax.fori_loop(0, CH // LANES, group_body, 0)
            pltpu.sync_copy(emv, em_hbm.at[pl.ds(base, CH)])

        return 0

    lax.fori_loop(0, nloop, chunk_body, 0)


def _edge_scores(col, row, gA, gB, w2p):
    e = col.shape[0]
    hid = gA.shape[1]
    nchunks = e // CH
    nloop = (nchunks + NW - 1) // NW
    mesh = plsc.VectorSubcoreMesh(core_axis_name="c", subcore_axis_name="s",
                                  num_cores=NC, num_subcores=NS)
    return pl.kernel(
        functools.partial(_score_body, nchunks, nloop),
        out_type=jax.ShapeDtypeStruct((e,), jnp.float32),
        mesh=mesh,
        compiler_params=pltpu.CompilerParams(needs_layout_passes=False,
                                             use_tc_tiling_on_sc=False),
        scratch_types=[
            pltpu.VMEM((CH,), jnp.int32),
            pltpu.VMEM((CH,), jnp.int32),
            pltpu.VMEM((CH, hid), jnp.float32),
            pltpu.VMEM((CH, hid), jnp.float32),
            pltpu.VMEM((80,), jnp.float32),
            pltpu.VMEM((LANES * LANES,), jnp.float32),
            pltpu.VMEM((CH,), jnp.float32),
            pltpu.SemaphoreType.DMA,
            pltpu.SemaphoreType.DMA,
        ],
    )(col, row, gA, gB, w2p)


def _thresh_body(kk_ref, em_ref, hard_ref, thr_ref):
    em = em_ref[...]
    bits = lax.bitcast_convert_type(em, jnp.int32)
    kk = kk_ref[0, 0]

    def it(_, lohi):
        lo, hi = lohi
        mid = lax.div(lo + hi, 2)
        cnt = jnp.sum((bits > mid).astype(jnp.int32))
        take_hi = cnt <= kk
        return (jnp.where(take_hi, lo, mid + 1), jnp.where(take_hi, mid, hi))

    _, hi = lax.fori_loop(0, 31, it, (jnp.int32(0), jnp.int32(0x3F800000)))
    thr = lax.bitcast_convert_type(hi, jnp.float32)
    thr_ref[0, 0] = thr
    hard_ref[...] = em > thr


def _threshold(em2d, kk):
    return pl.pallas_call(
        _thresh_body,
        in_specs=[
            pl.BlockSpec(memory_space=pltpu.SMEM),
            pl.BlockSpec(memory_space=pltpu.VMEM),
        ],
        out_specs=[
            pl.BlockSpec(memory_space=pltpu.VMEM),
            pl.BlockSpec(memory_space=pltpu.SMEM),
        ],
        out_shape=[
            jax.ShapeDtypeStruct(em2d.shape, jnp.bool_),
            jax.ShapeDtypeStruct((1, 1), jnp.float32),
        ],
    )(kk, em2d)


BAND = 16384
BAND_EPS = 8e-3


def _band_body(x_ref, w1_ref, b1_ref, w2_ref, b2_ref, o_ref):
    h = jax.nn.relu(jnp.dot(x_ref[...], w1_ref[...],
                            preferred_element_type=jnp.float32) + b1_ref[...])
    s = jnp.dot(h, w2_ref[...], preferred_element_type=jnp.float32) + b2_ref[...]
    o_ref[...] = jax.nn.sigmoid(s)


_BW = BAND // NW


def _bgather_body(cb_hbm, rb_hbm, emb_hbm, f1_hbm, f2_hbm, ib, fb, sem):
    cid = lax.axis_index("c")
    sid = lax.axis_index("s")
    wid = sid * NC + cid
    base = wid * _BW
    nch = _BW // CH

    pltpu.sync_copy(cb_hbm.at[pl.ds(wid * nch, nch)], ib)
    cps = [pltpu.async_copy(emb_hbm.at[ib.at[j]], fb.at[pl.ds(j * CH, CH)], sem)
           for j in range(nch)]
    for cp in cps:
        cp.wait()
    pltpu.sync_copy(fb, f1_hbm.at[pl.ds(base, _BW)])

    pltpu.sync_copy(rb_hbm.at[pl.ds(wid * nch, nch)], ib)
    cps = [pltpu.async_copy(emb_hbm.at[ib.at[j]], fb.at[pl.ds(j * CH, CH)], sem)
           for j in range(nch)]
    for cp in cps:
        cp.wait()
    pltpu.sync_copy(fb, f2_hbm.at[pl.ds(base, _BW)])


def _band_gather(cb_idx, rb_idx, embed):
    d = embed.shape[1]
    mesh = plsc.VectorSubcoreMesh(core_axis_name="c", subcore_axis_name="s",
                                  num_cores=NC, num_subcores=NS)
    return pl.kernel(
        _bgather_body,
        out_type=[
            jax.ShapeDtypeStruct((BAND, d), jnp.float32),
            jax.ShapeDtypeStruct((BAND, d), jnp.float32),
        ],
        mesh=mesh,
        compiler_params=pltpu.CompilerParams(needs_layout_passes=False,
                                             use_tc_tiling_on_sc=False),
        scratch_types=[
            pltpu.VMEM((_BW // CH, CH), jnp.int32),
            pltpu.VMEM((_BW, d), jnp.float32),
            pltpu.SemaphoreType.DMA,
        ],
    )(cb_idx.reshape(BAND // CH, CH), rb_idx.reshape(BAND // CH, CH), embed)


def _band_rescore(x, W1, b1, W2, b2):
    blk = 2048
    d2 = W1.shape[0]
    hid = W1.shape[1]
    out = pl.pallas_call(
        _band_body,
        grid=(BAND // blk,),
        in_specs=[
            pl.BlockSpec((blk, d2), lambda i: (i, 0)),
            pl.BlockSpec((d2, hid), lambda i: (0, 0)),
            pl.BlockSpec((1, hid), lambda i: (0, 0)),
            pl.BlockSpec((hid, 1), lambda i: (0, 0)),
            pl.BlockSpec((1, 1), lambda i: (0, 0)),
        ],
        out_specs=pl.BlockSpec((blk, 1), lambda i: (i, 0)),
        out_shape=jax.ShapeDtypeStruct((BAND, 1), jnp.float32),
    )(x, W1, b1.reshape(1, hid), W2, b2.reshape(1, 1))
    return out.reshape(-1)


def _nodesel_body(nchunks, nloop, n, col_hbm, row_hbm, em_hbm, thr_hbm, zero_hbm,
                  out_hbm, colv, rowv, emv, valv, thrv, fbuf, accsh):
    sid = lax.axis_index("s")
    cid = lax.axis_index("c")
    wid = sid + cid * 0

    pltpu.sync_copy(thr_hbm, thrv)
    tv = thrv[...]

    @pl.when(wid == 0)
    def _():
        pltpu.sync_copy(zero_hbm, accsh)

    plsc.subcore_barrier()

    def chunk_body(i, _):
        c = i * NS + wid

        @pl.when(c < nchunks)
        def _():
            base = c * CH
            pltpu.sync_copy(col_hbm.at[pl.ds(base, CH)], colv)
            pltpu.sync_copy(row_hbm.at[pl.ds(base, CH)], rowv)
            pltpu.sync_copy(em_hbm.at[pl.ds(base, CH)], emv)
            for g in range(CH // LANES):
                ev = emv[pl.ds(g * LANES, LANES)]
                valv[pl.ds(g * LANES, LANES)] = jnp.where(ev > tv, 1.0, 0.0)
            pltpu.sync_copy(valv, accsh.at[colv], add=True)
            pltpu.sync_copy(valv, accsh.at[rowv], add=True)

        return 0

    lax.fori_loop(0, nloop, chunk_body, 0)
    plsc.subcore_barrier()

    @pl.when(wid == 0)
    def _():
        pltpu.sync_copy(accsh, fbuf)

        def clamp_body(i, _):
            v = fbuf[pl.ds(i * LANES, LANES)]
            fbuf[pl.ds(i * LANES, LANES)] = jnp.where(v > 0.0, 1.0, 0.0)
            return 0

        lax.fori_loop(0, n // LANES, clamp_body, 0)
        pltpu.sync_copy(fbuf, out_hbm)


def _node_sel(col, row, em, thr16, n):
    e = col.shape[0]
    nchunks = e // CH
    nloop = (nchunks + NS - 1) // NS
    mesh = plsc.VectorSubcoreMesh(core_axis_name="c", subcore_axis_name="s",
                                  num_cores=1, num_subcores=NS)
    zeros = jnp.zeros((n,), jnp.float32)
    return pl.kernel(
        functools.partial(_nodesel_body, nchunks, nloop, n),
        out_type=jax.ShapeDtypeStruct((n,), jnp.float32),
        mesh=mesh,
        compiler_params=pltpu.CompilerParams(needs_layout_passes=False),
        scratch_types=[
            pltpu.VMEM((CH,), jnp.int32),
            pltpu.VMEM((CH,), jnp.int32),
            pltpu.VMEM((CH,), jnp.float32),
            pltpu.VMEM((CH,), jnp.float32),
            pltpu.VMEM((LANES,), jnp.float32),
            pltpu.VMEM((n,), jnp.float32),
            pltpu.VMEM_SHARED((n,), jnp.float32),
        ],
    )(col, row, em, thr16, zeros)


def kernel(embed, edge_index, W1, b1, W2, b2, top_k):
    n, d = embed.shape
    e = edge_index.shape[1]
    col = edge_index[0]
    row = edge_index[1]

    gA, gB = _node_tables(embed, W1, b1)
    w2p = jnp.concatenate([W2.reshape(-1), jnp.broadcast_to(b2.reshape(1), (16,))])
    em = _edge_scores(col, row, gA, gB, w2p)

    kk = jnp.minimum(jnp.asarray(top_k, jnp.int32), e - 1).reshape(1, 1)
    _, t0 = _threshold(em.reshape(e // CH, CH), kk)

    band = jnp.abs(em - t0.reshape(())) <= BAND_EPS
    bidx = jnp.where(band, size=BAND, fill_value=0)[0].astype(jnp.int32)
    f1, f2 = _band_gather(jnp.take(col, bidx), jnp.take(row, bidx), embed)
    em_band = _band_rescore(jnp.concatenate([f1, f2], axis=-1), W1, b1, W2, b2)
    em = em.at[bidx].set(em_band)

    hard2d, thr = _threshold(em.reshape(e // CH, CH), kk)
    hard = hard2d.reshape(e)

    thr16 = jnp.broadcast_to(thr.reshape(1), (16,))
    node_sel = _node_sel(col, row, em, thr16, n)
    return em, hard, node_sel
